# Initial kernel scaffold; baseline (speedup 1.0000x reference)
import jax
import jax.numpy as jnp
from jax.experimental import pallas as pl


def kernel(x, edge_index, batch, W1, b1, W2, b2, W3, b3, Wl1, bl1, Wl2, bl2):
    raise NotImplementedError("write your pallas kernel here")


# speedup vs baseline: 11.2230x; 11.2230x over previous
import functools

import jax
import jax.numpy as jnp
from jax import lax
from jax.experimental import pallas as pl
from jax.experimental.pallas import tpu as pltpu
from jax.experimental.pallas import tpu_sc as plsc

NC = 2
NS = 16
NW = NC * NS
C = 128

_mesh = lambda: plsc.VectorSubcoreMesh(core_axis_name="c", subcore_axis_name="s")


def _make_agg(n_nodes, h_dim, nchunk):
    zrows = -(-(n_nodes + 1) // (NS * 8)) * 8
    acc_rows = zrows * NS

    @functools.partial(
        pl.kernel,
        mesh=_mesh(),
        out_type=jax.ShapeDtypeStruct((NC * acc_rows, h_dim), jnp.float32),
        scratch_types=[
            pltpu.VMEM((C,), jnp.int32),
            pltpu.VMEM((C,), jnp.int32),
            pltpu.VMEM((C, h_dim), jnp.float32),
            pltpu.VMEM_SHARED((acc_rows, h_dim), jnp.float32),
            pltpu.SemaphoreType.DMA,
        ],
    )
    def agg(table_hbm, src_hbm, dst_hbm, zeros_hbm, out_hbm,
            src_v, dst_v, rows_v, acc_sh, sem):
        c = lax.axis_index("c")
        s = lax.axis_index("s")
        wid = s * NC + c
        pltpu.sync_copy(zeros_hbm, acc_sh.at[pl.ds(s * zrows, zrows)])
        plsc.subcore_barrier()

        def chunk(i, carry):
            base = (wid * nchunk + i) * C
            pltpu.sync_copy(src_hbm.at[pl.ds(base, C)], src_v)
            pltpu.sync_copy(dst_hbm.at[pl.ds(base, C)], dst_v)
            pltpu.async_copy(table_hbm.at[src_v], rows_v, sem).wait()
            pltpu.sync_copy(rows_v, acc_sh.at[dst_v], add=True)
            return carry

        lax.fori_loop(0, nchunk, chunk, 0)
        plsc.subcore_barrier()
        pltpu.sync_copy(acc_sh.at[pl.ds(s * zrows, zrows)],
                        out_hbm.at[pl.ds(c * acc_rows + s * zrows, zrows)])

    return agg, zrows, acc_rows


def _make_deg(n_nodes, nchunk):
    zrows = -(-(n_nodes + 1) // (NS * 8)) * 8
    acc_rows = zrows * NS

    @functools.partial(
        pl.kernel,
        mesh=_mesh(),
        out_type=jax.ShapeDtypeStruct((NC * acc_rows, 16), jnp.float32),
        scratch_types=[
            pltpu.VMEM((C,), jnp.int32),
            pltpu.VMEM((C, 16), jnp.float32),
            pltpu.VMEM_SHARED((acc_rows, 16), jnp.float32),
        ],
    )
    def deg(dst_hbm, ones_hbm, zeros_hbm, out_hbm, dst_v, ones_v, acc_sh):
        c = lax.axis_index("c")
        s = lax.axis_index("s")
        wid = s * NC + c
        pltpu.sync_copy(zeros_hbm, acc_sh.at[pl.ds(s * zrows, zrows)])
        pltpu.sync_copy(ones_hbm, ones_v)
        plsc.subcore_barrier()

        def chunk(i, carry):
            base = (wid * nchunk + i) * C
            pltpu.sync_copy(dst_hbm.at[pl.ds(base, C)], dst_v)
            pltpu.sync_copy(ones_v, acc_sh.at[dst_v], add=True)
            return carry

        lax.fori_loop(0, nchunk, chunk, 0)
        plsc.subcore_barrier()
        pltpu.sync_copy(acc_sh.at[pl.ds(s * zrows, zrows)],
                        out_hbm.at[pl.ds(c * acc_rows + s * zrows, zrows)])

    return deg, zrows, acc_rows


def _tc_first(x_ref, w_ref, dega_ref, degb_ref, out_ref):
    deg = dega_ref[:, :1] + degb_ref[:, :1]
    dis = lax.rsqrt(deg)
    out_ref[:] = jnp.dot(x_ref[:], w_ref[:],
                         preferred_element_type=jnp.float32) * dis


def _tc_mid(agga_ref, aggb_ref, dega_ref, degb_ref, b_ref, w_ref, out_ref):
    deg = dega_ref[:, :1] + degb_ref[:, :1]
    dis = lax.rsqrt(deg)
    h = jnp.maximum((agga_ref[:] + aggb_ref[:]) * dis + b_ref[:], 0.0)
    out_ref[:] = jnp.dot(h, w_ref[:], preferred_element_type=jnp.float32) * dis


def _tc_head(agga_ref, aggb_ref, dega_ref, degb_ref, b_ref, batch_ref,
             wl1_ref, bl1_ref, wl2_ref, bl2_ref, out_ref):
    n = agga_ref.shape[0]
    g = out_ref.shape[0]
    deg = dega_ref[:, :1] + degb_ref[:, :1]
    dis = lax.rsqrt(deg)
    h = jnp.maximum((agga_ref[:] + aggb_ref[:]) * dis + b_ref[:], 0.0)
    gids = lax.broadcasted_iota(jnp.int32, (g, n), 0)
    onehot = (batch_ref[:] == gids).astype(jnp.float32)
    sums = jnp.dot(onehot, h, preferred_element_type=jnp.float32)
    counts = jnp.sum(onehot, axis=1, keepdims=True)
    pooled = sums / jnp.maximum(counts, 1.0)
    hh = jnp.maximum(
        jnp.dot(pooled, wl1_ref[:], preferred_element_type=jnp.float32)
        + bl1_ref[:], 0.0)
    out_ref[:] = (jnp.dot(hh, wl2_ref[:], preferred_element_type=jnp.float32)
                  + bl2_ref[:])


def kernel(x, edge_index, batch, W1, b1, W2, b2, W3, b3, Wl1, bl1, Wl2, bl2):
    n, d = x.shape
    h_dim = W1.shape[1]
    g = 64
    e = edge_index.shape[1]
    e_tot = e + n
    nchunk = -(-e_tot // (NW * C))
    e_pad = NW * nchunk * C

    loop = jnp.arange(n, dtype=jnp.int32)
    src = jnp.concatenate([
        edge_index[0].astype(jnp.int32), loop,
        jnp.zeros((e_pad - e_tot,), jnp.int32)])
    dst = jnp.concatenate([
        edge_index[1].astype(jnp.int32), loop,
        jnp.full((e_pad - e_tot,), n, jnp.int32)])

    agg, azr, arows = _make_agg(n, h_dim, nchunk)
    deg, dzr, drows = _make_deg(n, nchunk)

    zeros_h = jnp.zeros((azr, h_dim), jnp.float32)
    zeros_d = jnp.zeros((dzr, 16), jnp.float32)
    ones_d = jnp.ones((C, 16), jnp.float32)

    deg2 = deg(dst, ones_d, zeros_d)
    dega, degb = deg2[:n], deg2[drows:drows + n]

    tc_io = functools.partial(pl.pallas_call,
                              out_shape=jax.ShapeDtypeStruct((n, h_dim),
                                                             jnp.float32))

    hw = tc_io(_tc_first)(x, W1, dega, degb)
    for b, w in ((b1, W2), (b2, W3)):
        a2 = agg(hw, src, dst, zeros_h)
        hw = tc_io(_tc_mid)(a2[:n], a2[arows:arows + n], dega, degb,
                            b.reshape(1, h_dim), w)
    a2 = agg(hw, src, dst, zeros_h)
    out = pl.pallas_call(
        _tc_head,
        out_shape=jax.ShapeDtypeStruct((g, 1), jnp.float32),
    )(a2[:n], a2[arows:arows + n], dega, degb, b3.reshape(1, h_dim),
      batch.reshape(1, n).astype(jnp.int32), Wl1, bl1.reshape(1, -1),
      Wl2, bl2.reshape(1, 1))
    return out.reshape(g)
